# Initial kernel scaffold; baseline (speedup 1.0000x reference)
import jax
import jax.numpy as jnp
from jax.experimental import pallas as pl


def kernel(x, edge_index, W, b):
    raise NotImplementedError("write your pallas kernel here")


# speedup vs baseline: 18.0409x; 18.0409x over previous
import functools

import jax
import jax.numpy as jnp
from jax import lax
from jax.experimental import pallas as pl
from jax.experimental.pallas import tpu as pltpu
from jax.experimental.pallas import tpu_sc as plsc

N = 10000
IN_CH = 128
OUT_CH = 128
E = 320000

NC = 2
NS = 16
NW = NC * NS

BLK = 128
NBLK = -(-E // (NW * BLK))
E_PAD = NW * NBLK * BLK
PAD_N = 10240
ROWS_PER_SUB = PAD_N // NS

_mesh = plsc.VectorSubcoreMesh(core_axis_name="c", subcore_axis_name="s")


def _zero_2d(ref, rows, cols):
    z = jnp.zeros((16,), jnp.float32)

    def body(r, _):
        for cc in range(cols // 16):
            ref[r, pl.ds(cc * 16, 16)] = z
        return 0

    lax.fori_loop(0, rows, body, 0)


def _deg_body(dst_hbm, out_hbm, dstv, zb, onesv, hist_sh):
    c = lax.axis_index("c")
    s = lax.axis_index("s")
    wid = s * NC + c

    def zbody(k, _):
        zb[pl.ds(k * 16, 16)] = jnp.zeros((16,), jnp.float32)
        return 0

    lax.fori_loop(0, ROWS_PER_SUB // 16, zbody, 0)
    for k in range(BLK // 16):
        onesv[pl.ds(k * 16, 16)] = jnp.ones((16,), jnp.float32)

    pltpu.sync_copy(zb, hist_sh.at[pl.ds(s * ROWS_PER_SUB, ROWS_PER_SUB)])
    pltpu.sync_copy(dst_hbm.at[wid], dstv)
    plsc.subcore_barrier()

    def body(j, _):
        pltpu.sync_copy(onesv, hist_sh.at[dstv.at[j]], add=True)
        return 0

    lax.fori_loop(0, NBLK, body, 0)
    plsc.subcore_barrier()
    pltpu.sync_copy(
        hist_sh.at[pl.ds(s * ROWS_PER_SUB, ROWS_PER_SUB)],
        out_hbm.at[c, pl.ds(s * ROWS_PER_SUB, ROWS_PER_SUB)],
    )


_deg_kernel = functools.partial(
    pl.kernel,
    out_type=jax.ShapeDtypeStruct((NC, PAD_N), jnp.float32),
    mesh=_mesh,
    scratch_types=[
        pltpu.VMEM((NBLK, BLK), jnp.int32),
        pltpu.VMEM((ROWS_PER_SUB,), jnp.float32),
        pltpu.VMEM((BLK,), jnp.float32),
        pltpu.VMEM_SHARED((PAD_N,), jnp.float32),
    ],
)(_deg_body)


def _agg_body(y_hbm, src_hbm, dst_hbm, out_hbm, srcv, dstv, rb0, acc_sh, sem0):
    c = lax.axis_index("c")
    s = lax.axis_index("s")
    wid = s * NC + c

    _zero_2d(rb0, BLK, OUT_CH)
    for t in range(ROWS_PER_SUB // BLK):
        pltpu.sync_copy(rb0, acc_sh.at[pl.ds(s * ROWS_PER_SUB + t * BLK, BLK)])
    pltpu.sync_copy(src_hbm.at[wid], srcv)
    pltpu.sync_copy(dst_hbm.at[wid], dstv)
    plsc.subcore_barrier()

    def body(j, _):
        pltpu.async_copy(y_hbm.at[srcv.at[j]], rb0, sem0).wait()
        pltpu.sync_copy(rb0, acc_sh.at[dstv.at[j]], add=True)
        return 0

    lax.fori_loop(0, NBLK, body, 0)
    plsc.subcore_barrier()
    pltpu.sync_copy(
        acc_sh.at[pl.ds(s * ROWS_PER_SUB, ROWS_PER_SUB)],
        out_hbm.at[c, pl.ds(s * ROWS_PER_SUB, ROWS_PER_SUB)],
    )


_agg_kernel = functools.partial(
    pl.kernel,
    out_type=jax.ShapeDtypeStruct((NC, PAD_N, OUT_CH), jnp.float32),
    mesh=_mesh,
    scratch_types=[
        pltpu.VMEM((NBLK, BLK), jnp.int32),
        pltpu.VMEM((NBLK, BLK), jnp.int32),
        pltpu.VMEM((BLK, OUT_CH), jnp.float32),
        pltpu.VMEM_SHARED((PAD_N, OUT_CH), jnp.float32),
        pltpu.SemaphoreType.DMA,
    ],
)(_agg_body)


def _mm_body(x_ref, w_ref, p0_ref, p1_ref, y_ref, dis_ref):
    xw = jnp.dot(x_ref[...], w_ref[...], preferred_element_type=jnp.float32)
    deg = p0_ref[...] + p1_ref[...] + 1.0
    dis = lax.rsqrt(deg)
    dis_ref[...] = dis
    y_ref[...] = dis * xw


def _mm_kernel(x, W, p0c, p1c):
    return pl.pallas_call(
        _mm_body,
        out_shape=(
            jax.ShapeDtypeStruct((N, OUT_CH), jnp.float32),
            jax.ShapeDtypeStruct((N, 1), jnp.float32),
        ),
    )(x, W, p0c, p1c)


def _fin_body(a0_ref, a1_ref, y_ref, dis_ref, b_ref, o_ref):
    acc = a0_ref[...] + a1_ref[...] + y_ref[...]
    o_ref[...] = jnp.maximum(dis_ref[...] * acc + b_ref[...], 0.0)


def _fin_kernel(a0, a1, y, dis, b):
    return pl.pallas_call(
        _fin_body,
        out_shape=jax.ShapeDtypeStruct((N, OUT_CH), jnp.float32),
    )(a0, a1, y, dis, b)


def kernel(x, edge_index, W, b):
    src = edge_index[0].astype(jnp.int32)
    dst = edge_index[1].astype(jnp.int32)
    pad = E_PAD - E
    src_p = jnp.concatenate([src, jnp.zeros((pad,), jnp.int32)])
    dst_p = jnp.concatenate([dst, jnp.full((pad,), N, jnp.int32)])
    src3 = src_p.reshape(NW, NBLK, BLK)
    dst3 = dst_p.reshape(NW, NBLK, BLK)

    deg_parts = _deg_kernel(dst3)
    p0c = deg_parts[0, :N].reshape(N, 1)
    p1c = deg_parts[1, :N].reshape(N, 1)
    y, dis = _mm_kernel(x, W, p0c, p1c)
    acc_parts = _agg_kernel(y, src3, dst3)
    out = _fin_kernel(acc_parts[0, :N], acc_parts[1, :N], y, dis, b)
    return out
